# Initial kernel scaffold; baseline (speedup 1.0000x reference)
import jax
import jax.numpy as jnp
from jax.experimental import pallas as pl


def kernel(sentence, word_chars, char_lens, word_emb, char_emb, Wih_c, Whh_c, b_c, Wih_w, Whh_w, b_w, Wout, bout):
    raise NotImplementedError("write your pallas kernel here")


# speedup vs baseline: 1.6661x; 1.6661x over previous
import functools

import jax
import jax.numpy as jnp
from jax import lax
from jax.experimental import pallas as pl
from jax.experimental.pallas import tpu as pltpu

_S, _LC = 8192, 16
_D, _DC = 256, 64
_HC, _H, _T = 128, 256, 17
_TP = 128


def _gather_kernel(sent_ref, wemb_ref, o_ref):
    o_ref[...] = wemb_ref[...]


def _gather_rows(sentence, word_emb):
    s = sentence.shape[0]
    v, d = word_emb.shape
    out = pl.pallas_call(
        _gather_kernel,
        grid_spec=pltpu.PrefetchScalarGridSpec(
            num_scalar_prefetch=1,
            grid=(s,),
            in_specs=[pl.BlockSpec((1, 1, d), lambda i, sent: (sent[i], 0, 0))],
            out_specs=pl.BlockSpec((1, 1, d), lambda i, sent: (i, 0, 0)),
        ),
        out_shape=jax.ShapeDtypeStruct((s, 1, d), jnp.float32),
        compiler_params=pltpu.CompilerParams(
            dimension_semantics=("arbitrary",),
        ),
        name="wemb_gather",
    )(sentence.astype(jnp.int32), word_emb.reshape(v, 1, d))
    return out.reshape(s, d)


def _sig(x):
    return jax.nn.sigmoid(x)


def _main_kernel(lens_ref, chars_ref, we_ref, cemb_ref, wihcT_ref, bc_ref,
                 whhcT_ref, wweT_ref, whcT_ref, whhwT_ref, bw_ref,
                 woutT_ref, bout_ref, out_ref, cew_ref):
    cew_ref[...] = jnp.dot(cemb_ref[...], wihcT_ref[...],
                           preferred_element_type=jnp.float32) + bc_ref[...]

    def char_body(base, t, st):
        h, c = st
        cid = chars_ref[base + t]
        gates = cew_ref[pl.ds(cid, 1), :] + jnp.dot(
            h, whhcT_ref[...], preferred_element_type=jnp.float32)
        i = gates[:, 0 * _HC:1 * _HC]
        f = gates[:, 1 * _HC:2 * _HC]
        g = gates[:, 2 * _HC:3 * _HC]
        o = gates[:, 3 * _HC:4 * _HC]
        c2 = _sig(f) * c + _sig(i) * jnp.tanh(g)
        h2 = _sig(o) * jnp.tanh(c2)
        return (h2, c2)

    def word_body(w, carry):
        hc, cc, hw, cw = carry
        length = lens_ref[w]
        base = w * _LC
        hc, cc = lax.fori_loop(0, length,
                               functools.partial(char_body, base), (hc, cc))
        we_row = we_ref[pl.ds(w, 1), :]
        wg = (jnp.dot(we_row, wweT_ref[...], preferred_element_type=jnp.float32)
              + bw_ref[...]
              + jnp.dot(hc, whcT_ref[...], preferred_element_type=jnp.float32)
              + jnp.dot(hw, whhwT_ref[...], preferred_element_type=jnp.float32))
        iw = wg[:, 0 * _H:1 * _H]
        fw = wg[:, 1 * _H:2 * _H]
        gw = wg[:, 2 * _H:3 * _H]
        ow = wg[:, 3 * _H:4 * _H]
        cw = _sig(fw) * cw + _sig(iw) * jnp.tanh(gw)
        hw = _sig(ow) * jnp.tanh(cw)
        logit = jnp.dot(hw, woutT_ref[...],
                        preferred_element_type=jnp.float32) + bout_ref[...]
        out_ref[pl.ds(w, 1)] = logit.reshape(1, 1, _TP)
        return (hc, cc, hw, cw)

    z1 = jnp.zeros((1, _HC), jnp.float32)
    z2 = jnp.zeros((1, _H), jnp.float32)
    lax.fori_loop(0, _S, word_body, (z1, z1, z2, z2))


def kernel(sentence, word_chars, char_lens, word_emb, char_emb,
           Wih_c, Whh_c, b_c, Wih_w, Whh_w, b_w, Wout, bout):
    we = _gather_rows(sentence, word_emb)

    lens = char_lens.astype(jnp.int32)
    chars_flat = word_chars.reshape(-1).astype(jnp.int32)

    wihcT = Wih_c.T
    whhcT = Whh_c.T
    bc2 = b_c[None, :]
    wweT = Wih_w[:, :_D].T
    whcT = Wih_w[:, _D:].T
    whhwT = Whh_w.T
    bw2 = b_w[None, :]
    woutT = jnp.zeros((_H, _TP), jnp.float32).at[:, :_T].set(Wout.T)
    boutp = jnp.zeros((1, _TP), jnp.float32).at[:, :_T].set(bout[None, :])

    smem = pl.BlockSpec(memory_space=pltpu.SMEM)
    vmem = pl.BlockSpec(memory_space=pltpu.VMEM)
    out = pl.pallas_call(
        _main_kernel,
        in_specs=[smem, smem] + [vmem] * 11,
        out_specs=vmem,
        out_shape=jax.ShapeDtypeStruct((_S, 1, _TP), jnp.float32),
        scratch_shapes=[pltpu.VMEM((128, 4 * _HC), jnp.float32)],
        compiler_params=pltpu.CompilerParams(
            vmem_limit_bytes=50 * 1024 * 1024,
        ),
        name="dual_tagger_chain",
    )(lens, chars_flat, we, char_emb, wihcT, bc2, whhcT, wweT, whcT,
      whhwT, bw2, woutT, boutp)
    return out.reshape(_S, _TP)[:, :_T]
